# baseline (speedup 1.0000x reference)
import functools

import jax
import jax.numpy as jnp
from jax import lax
from jax.experimental import pallas as pl
from jax.experimental.pallas import tpu as pltpu
from jax.experimental.pallas import tpu_sc as plsc

B = 16384
F = 26
V = 100000
D = 16

NC, NS, L = 2, 16, 16
NW = NC * NS
NB = B // NW
JG = NB // L

VP = 102400
CHUNKS = ((0, 7), (7, 7), (14, 6), (20, 6))


def _fold_body(dnn_ref, lin_ref, w_ref, out_ref, *, w_base):
    f = pl.program_id(0)
    d = dnn_ref[0]
    w = w_ref[w_base + f]
    dot = jnp.dot(w[None, :], d, preferred_element_type=jnp.float32)
    out_ref[...] = lin_ref[0, 0, :] + dot[0]


def _sc_gather_body(xt_hbm, comb_hbm, *args, f_base, f_cnt, first, final):
    if first:
        out_hbm, xl, idxb, vals, outl, sem = args
    else:
        part_hbm, out_hbm, xl, idxb, vals, outl, sem = args
    wid = lax.axis_index("s") * NC + lax.axis_index("c")
    base = wid * NB

    pltpu.sync_copy(xt_hbm.at[pl.ds(f_base, f_cnt), pl.ds(base, NB)], xl)

    def build_f(f, _):
        off = f * VP
        for jc in range(JG):
            v = xl[f, pl.ds(jc * L, L)]
            idxb[pl.ds(f * NB + jc * L, L)] = v + off
        return _
    lax.fori_loop(0, f_cnt, build_f, None)

    pltpu.async_copy(comb_hbm.at[idxb], vals, sem).wait()

    if first:
        zero = jnp.zeros((L,), jnp.float32)
        def init(jg, _):
            outl[pl.ds(jg * L, L)] = zero
            return _
        lax.fori_loop(0, JG, init, None)
    else:
        pltpu.sync_copy(part_hbm.at[pl.ds(base, NB)], outl)

    def group(jg, _):
        acc = outl[pl.ds(jg * L, L)]
        for f in range(f_cnt):
            acc = acc + vals[pl.ds(f * NB + jg * L, L)]
        if final:
            acc = 1.0 / (1.0 + jnp.exp(-acc))
        outl[pl.ds(jg * L, L)] = acc
        return _
    lax.fori_loop(0, JG, group, None)

    pltpu.sync_copy(outl, out_hbm.at[pl.ds(base, NB)])


def _make_fold(w_base, f_cnt):
    return pl.pallas_call(
        functools.partial(_fold_body, w_base=w_base),
        grid=(f_cnt, 1),
        in_specs=[
            pl.BlockSpec((1, D, VP), lambda f, i: (w_base + f, 0, i)),
            pl.BlockSpec((1, 1, VP), lambda f, i: (w_base + f, 0, i)),
            pl.BlockSpec((F, D), lambda f, i: (0, 0)),
        ],
        out_specs=pl.BlockSpec((VP,), lambda f, i: (f,)),
        out_shape=jax.ShapeDtypeStruct((f_cnt * VP,), jnp.float32),
    )


def _make_gather(f_base, f_cnt, first, final):
    mesh = plsc.VectorSubcoreMesh(core_axis_name="c", subcore_axis_name="s",
                                  num_cores=NC, num_subcores=NS)
    return pl.kernel(
        functools.partial(_sc_gather_body, f_base=f_base, f_cnt=f_cnt,
                          first=first, final=final),
        out_type=jax.ShapeDtypeStruct((B,), jnp.float32),
        mesh=mesh,
        compiler_params=pltpu.CompilerParams(
            needs_layout_passes=False, use_tc_tiling_on_sc=False),
        scratch_types=[
            pltpu.VMEM((f_cnt, NB), jnp.int32),
            pltpu.VMEM((f_cnt * NB,), jnp.int32),
            pltpu.VMEM((f_cnt * NB,), jnp.float32),
            pltpu.VMEM((NB,), jnp.float32),
            pltpu.SemaphoreType.DMA,
        ],
    )


@jax.jit
def kernel(X, linear_tables, dnn_tables, W_out):
    xt = X.astype(jnp.int32).T
    dnn_t = jnp.transpose(dnn_tables, (0, 2, 1))
    lin_t = jnp.transpose(linear_tables, (0, 2, 1))
    w = W_out.reshape(F, D)

    part = None
    for k, (fb, fc) in enumerate(CHUNKS):
        comb = _make_fold(fb, fc)(dnn_t, lin_t, w)
        gather = _make_gather(fb, fc, first=(k == 0),
                              final=(k == len(CHUNKS) - 1))
        if k == 0:
            part = gather(xt, comb)
        else:
            part = gather(xt, comb, part)
    return part.reshape(B, 1)


# speedup vs baseline: 1.0272x; 1.0272x over previous
import functools

import jax
import jax.numpy as jnp
from jax import lax
from jax.experimental import pallas as pl
from jax.experimental.pallas import tpu as pltpu
from jax.experimental.pallas import tpu_sc as plsc

B = 16384
F = 26
V = 100000
D = 16

NC, NS, L = 2, 16, 16
NW = NC * NS
NB = B // NW
JG = NB // L

VP = 102400
CHUNKS = ((0, 13), (13, 13))


def _fold_body(dnn_ref, lin_ref, w_ref, out_ref, *, w_base):
    f = pl.program_id(0)
    d = dnn_ref[0]
    w = w_ref[w_base + f]
    dot = jnp.dot(w[None, :], d, preferred_element_type=jnp.float32)
    out_ref[...] = lin_ref[0, 0, :] + dot[0]


def _sc_gather_body(xt_hbm, comb_hbm, *args, f_base, f_cnt, first, final):
    if first:
        out_hbm, xl, idxb, vals, outl, sem = args
    else:
        part_hbm, out_hbm, xl, idxb, vals, outl, sem = args
    wid = lax.axis_index("s") * NC + lax.axis_index("c")
    base = wid * NB

    pltpu.sync_copy(xt_hbm.at[pl.ds(f_base, f_cnt), pl.ds(base, NB)], xl)

    def build_f(f, _):
        off = f * VP
        for jc in range(JG):
            v = xl[f, pl.ds(jc * L, L)]
            idxb[pl.ds(f * NB + jc * L, L)] = v + off
        return _
    lax.fori_loop(0, f_cnt, build_f, None)

    pltpu.async_copy(comb_hbm.at[idxb], vals, sem).wait()

    if first:
        zero = jnp.zeros((L,), jnp.float32)
        def init(jg, _):
            outl[pl.ds(jg * L, L)] = zero
            return _
        lax.fori_loop(0, JG, init, None)
    else:
        pltpu.sync_copy(part_hbm.at[pl.ds(base, NB)], outl)

    def group(jg, _):
        acc = outl[pl.ds(jg * L, L)]
        for f in range(f_cnt):
            acc = acc + vals[pl.ds(f * NB + jg * L, L)]
        if final:
            acc = 1.0 / (1.0 + jnp.exp(-acc))
        outl[pl.ds(jg * L, L)] = acc
        return _
    lax.fori_loop(0, JG, group, None)

    pltpu.sync_copy(outl, out_hbm.at[pl.ds(base, NB)])


def _make_fold(w_base, f_cnt):
    return pl.pallas_call(
        functools.partial(_fold_body, w_base=w_base),
        grid=(f_cnt, 1),
        in_specs=[
            pl.BlockSpec((1, D, VP), lambda f, i: (w_base + f, 0, i)),
            pl.BlockSpec((1, 1, VP), lambda f, i: (w_base + f, 0, i)),
            pl.BlockSpec((F, D), lambda f, i: (0, 0)),
        ],
        out_specs=pl.BlockSpec((VP,), lambda f, i: (f,)),
        out_shape=jax.ShapeDtypeStruct((f_cnt * VP,), jnp.float32),
    )


def _make_gather(f_base, f_cnt, first, final):
    mesh = plsc.VectorSubcoreMesh(core_axis_name="c", subcore_axis_name="s",
                                  num_cores=NC, num_subcores=NS)
    return pl.kernel(
        functools.partial(_sc_gather_body, f_base=f_base, f_cnt=f_cnt,
                          first=first, final=final),
        out_type=jax.ShapeDtypeStruct((B,), jnp.float32),
        mesh=mesh,
        compiler_params=pltpu.CompilerParams(
            needs_layout_passes=False, use_tc_tiling_on_sc=False),
        scratch_types=[
            pltpu.VMEM((f_cnt, NB), jnp.int32),
            pltpu.VMEM((f_cnt * NB,), jnp.int32),
            pltpu.VMEM((f_cnt * NB,), jnp.float32),
            pltpu.VMEM((NB,), jnp.float32),
            pltpu.SemaphoreType.DMA,
        ],
    )


@jax.jit
def kernel(X, linear_tables, dnn_tables, W_out):
    xt = X.astype(jnp.int32).T
    dnn_t = jnp.transpose(dnn_tables, (0, 2, 1))
    lin_t = jnp.transpose(linear_tables, (0, 2, 1))
    w = W_out.reshape(F, D)

    part = None
    for k, (fb, fc) in enumerate(CHUNKS):
        comb = _make_fold(fb, fc)(dnn_t, lin_t, w)
        gather = _make_gather(fb, fc, first=(k == 0),
                              final=(k == len(CHUNKS) - 1))
        if k == 0:
            part = gather(xt, comb)
        else:
            part = gather(xt, comb, part)
    return part.reshape(B, 1)


# speedup vs baseline: 1.0389x; 1.0114x over previous
import functools

import jax
import jax.numpy as jnp
from jax import lax
from jax.experimental import pallas as pl
from jax.experimental.pallas import tpu as pltpu
from jax.experimental.pallas import tpu_sc as plsc

B = 16384
F = 26
V = 100000
D = 16

NC, NS, L = 2, 16, 16
NW = NC * NS
NB = B // NW
JG = NB // L

VP = 102400
CHUNKS = ((0, 13), (13, 13))


def _fold_body(dnn_ref, lin_ref, w_ref, out_ref, *, w_base):
    f = pl.program_id(0)
    d = dnn_ref[0]
    w = w_ref[w_base + f]
    dot = jnp.dot(w[None, :], d, preferred_element_type=jnp.float32)
    out_ref[...] = lin_ref[0, 0, :] + dot[0]


def _sc_gather_body(xt_hbm, comb_hbm, *args, f_base, f_cnt, first, final):
    if first:
        out_hbm, xl, idx1, idx2, vals1, vals2, outl, sem1, sem2 = args
    else:
        (part_hbm, out_hbm, xl, idx1, idx2, vals1, vals2, outl,
         sem1, sem2) = args
    h1 = f_cnt // 2
    h2 = f_cnt - h1
    wid = lax.axis_index("s") * NC + lax.axis_index("c")
    base = wid * NB

    pltpu.sync_copy(xt_hbm.at[pl.ds(f_base, f_cnt), pl.ds(base, NB)], xl)

    def build(f_lo, h_cnt, idxref):
        def build_f(f, _):
            off = (f_lo + f) * VP
            for jc in range(JG):
                v = xl[f_lo + f, pl.ds(jc * L, L)]
                idxref[pl.ds(f * NB + jc * L, L)] = v + off
            return _
        lax.fori_loop(0, h_cnt, build_f, None)

    build(0, h1, idx1)
    cp1 = pltpu.async_copy(comb_hbm.at[idx1], vals1, sem1)
    build(h1, h2, idx2)
    cp2 = pltpu.async_copy(comb_hbm.at[idx2], vals2, sem2)

    if not first:
        pltpu.sync_copy(part_hbm.at[pl.ds(base, NB)], outl)

    cp1.wait()

    def group1(jg, _):
        if first:
            acc = jnp.zeros((L,), jnp.float32)
        else:
            acc = outl[pl.ds(jg * L, L)]
        for f in range(h1):
            acc = acc + vals1[pl.ds(f * NB + jg * L, L)]
        outl[pl.ds(jg * L, L)] = acc
        return _
    lax.fori_loop(0, JG, group1, None)

    cp2.wait()

    def group2(jg, _):
        acc = outl[pl.ds(jg * L, L)]
        for f in range(h2):
            acc = acc + vals2[pl.ds(f * NB + jg * L, L)]
        if final:
            acc = 1.0 / (1.0 + jnp.exp(-acc))
        outl[pl.ds(jg * L, L)] = acc
        return _
    lax.fori_loop(0, JG, group2, None)

    pltpu.sync_copy(outl, out_hbm.at[pl.ds(base, NB)])


def _make_fold(w_base, f_cnt):
    return pl.pallas_call(
        functools.partial(_fold_body, w_base=w_base),
        grid=(f_cnt, 1),
        in_specs=[
            pl.BlockSpec((1, D, VP), lambda f, i: (w_base + f, 0, i)),
            pl.BlockSpec((1, 1, VP), lambda f, i: (w_base + f, 0, i)),
            pl.BlockSpec((F, D), lambda f, i: (0, 0)),
        ],
        out_specs=pl.BlockSpec((VP,), lambda f, i: (f,)),
        out_shape=jax.ShapeDtypeStruct((f_cnt * VP,), jnp.float32),
    )


def _make_gather(f_base, f_cnt, first, final):
    mesh = plsc.VectorSubcoreMesh(core_axis_name="c", subcore_axis_name="s",
                                  num_cores=NC, num_subcores=NS)
    h1 = f_cnt // 2
    h2 = f_cnt - h1
    return pl.kernel(
        functools.partial(_sc_gather_body, f_base=f_base, f_cnt=f_cnt,
                          first=first, final=final),
        out_type=jax.ShapeDtypeStruct((B,), jnp.float32),
        mesh=mesh,
        compiler_params=pltpu.CompilerParams(
            needs_layout_passes=False, use_tc_tiling_on_sc=False),
        scratch_types=[
            pltpu.VMEM((f_cnt, NB), jnp.int32),
            pltpu.VMEM((h1 * NB,), jnp.int32),
            pltpu.VMEM((h2 * NB,), jnp.int32),
            pltpu.VMEM((h1 * NB,), jnp.float32),
            pltpu.VMEM((h2 * NB,), jnp.float32),
            pltpu.VMEM((NB,), jnp.float32),
            pltpu.SemaphoreType.DMA,
            pltpu.SemaphoreType.DMA,
        ],
    )


@jax.jit
def kernel(X, linear_tables, dnn_tables, W_out):
    xt = X.astype(jnp.int32).T
    dnn_t = jnp.transpose(dnn_tables, (0, 2, 1))
    lin_t = jnp.transpose(linear_tables, (0, 2, 1))
    w = W_out.reshape(F, D)

    part = None
    for k, (fb, fc) in enumerate(CHUNKS):
        comb = _make_fold(fb, fc)(dnn_t, lin_t, w)
        gather = _make_gather(fb, fc, first=(k == 0),
                              final=(k == len(CHUNKS) - 1))
        if k == 0:
            part = gather(xt, comb)
        else:
            part = gather(xt, comb, part)
    return part.reshape(B, 1)


# speedup vs baseline: 1.0432x; 1.0041x over previous
import functools

import jax
import jax.numpy as jnp
from jax import lax
from jax.experimental import pallas as pl
from jax.experimental.pallas import tpu as pltpu
from jax.experimental.pallas import tpu_sc as plsc

B = 16384
F = 26
V = 100000
D = 16

NC, NS, L = 2, 16, 16
NW = NC * NS
NB = B // NW
JG = NB // L

VP = 102400
CHUNKS = ((0, 17), (17, 9))


def _fold_body(dnn_ref, lin_ref, w_ref, out_ref, *, w_base):
    f = pl.program_id(0)
    d = dnn_ref[0]
    w = w_ref[w_base + f]
    dot = jnp.dot(w[None, :], d, preferred_element_type=jnp.float32)
    out_ref[...] = lin_ref[0, 0, :] + dot[0]


def _sc_gather_body(xt_hbm, comb_hbm, *args, f_base, f_cnt, first, final):
    if first:
        out_hbm, xl, idx1, idx2, vals1, vals2, outl, sem1, sem2 = args
    else:
        (part_hbm, out_hbm, xl, idx1, idx2, vals1, vals2, outl,
         sem1, sem2) = args
    h1 = f_cnt // 2
    h2 = f_cnt - h1
    wid = lax.axis_index("s") * NC + lax.axis_index("c")
    base = wid * NB

    pltpu.sync_copy(xt_hbm.at[pl.ds(f_base, f_cnt), pl.ds(base, NB)], xl)

    def build(f_lo, h_cnt, idxref):
        def build_f(f, _):
            off = (f_lo + f) * VP
            for jc in range(JG):
                v = xl[f_lo + f, pl.ds(jc * L, L)]
                idxref[pl.ds(f * NB + jc * L, L)] = v + off
            return _
        lax.fori_loop(0, h_cnt, build_f, None)

    build(0, h1, idx1)
    cp1 = pltpu.async_copy(comb_hbm.at[idx1], vals1, sem1)
    build(h1, h2, idx2)
    cp2 = pltpu.async_copy(comb_hbm.at[idx2], vals2, sem2)

    if not first:
        pltpu.sync_copy(part_hbm.at[pl.ds(base, NB)], outl)

    cp1.wait()

    def group1(jg, _):
        if first:
            acc = jnp.zeros((L,), jnp.float32)
        else:
            acc = outl[pl.ds(jg * L, L)]
        for f in range(h1):
            acc = acc + vals1[pl.ds(f * NB + jg * L, L)]
        outl[pl.ds(jg * L, L)] = acc
        return _
    lax.fori_loop(0, JG, group1, None)

    cp2.wait()

    def group2(jg, _):
        acc = outl[pl.ds(jg * L, L)]
        for f in range(h2):
            acc = acc + vals2[pl.ds(f * NB + jg * L, L)]
        if final:
            acc = 1.0 / (1.0 + jnp.exp(-acc))
        outl[pl.ds(jg * L, L)] = acc
        return _
    lax.fori_loop(0, JG, group2, None)

    pltpu.sync_copy(outl, out_hbm.at[pl.ds(base, NB)])


def _make_fold(w_base, f_cnt):
    return pl.pallas_call(
        functools.partial(_fold_body, w_base=w_base),
        grid=(f_cnt, 1),
        in_specs=[
            pl.BlockSpec((1, D, VP), lambda f, i: (w_base + f, 0, i)),
            pl.BlockSpec((1, 1, VP), lambda f, i: (w_base + f, 0, i)),
            pl.BlockSpec((F, D), lambda f, i: (0, 0)),
        ],
        out_specs=pl.BlockSpec((VP,), lambda f, i: (f,)),
        out_shape=jax.ShapeDtypeStruct((f_cnt * VP,), jnp.float32),
    )


def _make_gather(f_base, f_cnt, first, final):
    mesh = plsc.VectorSubcoreMesh(core_axis_name="c", subcore_axis_name="s",
                                  num_cores=NC, num_subcores=NS)
    h1 = f_cnt // 2
    h2 = f_cnt - h1
    return pl.kernel(
        functools.partial(_sc_gather_body, f_base=f_base, f_cnt=f_cnt,
                          first=first, final=final),
        out_type=jax.ShapeDtypeStruct((B,), jnp.float32),
        mesh=mesh,
        compiler_params=pltpu.CompilerParams(
            needs_layout_passes=False, use_tc_tiling_on_sc=False),
        scratch_types=[
            pltpu.VMEM((f_cnt, NB), jnp.int32),
            pltpu.VMEM((h1 * NB,), jnp.int32),
            pltpu.VMEM((h2 * NB,), jnp.int32),
            pltpu.VMEM((h1 * NB,), jnp.float32),
            pltpu.VMEM((h2 * NB,), jnp.float32),
            pltpu.VMEM((NB,), jnp.float32),
            pltpu.SemaphoreType.DMA,
            pltpu.SemaphoreType.DMA,
        ],
    )


@jax.jit
def kernel(X, linear_tables, dnn_tables, W_out):
    xt = X.astype(jnp.int32).T
    dnn_t = jnp.transpose(dnn_tables, (0, 2, 1))
    lin_t = jnp.transpose(linear_tables, (0, 2, 1))
    w = W_out.reshape(F, D)

    part = None
    for k, (fb, fc) in enumerate(CHUNKS):
        comb = _make_fold(fb, fc)(dnn_t, lin_t, w)
        gather = _make_gather(fb, fc, first=(k == 0),
                              final=(k == len(CHUNKS) - 1))
        if k == 0:
            part = gather(xt, comb)
        else:
            part = gather(xt, comb, part)
    return part.reshape(B, 1)
